# baseline (speedup 1.0000x reference)
import functools

import jax
import jax.numpy as jnp
from jax import lax
from jax.experimental import pallas as pl
from jax.experimental.pallas import tpu as pltpu
from jax.experimental.pallas import tpu_sc as plsc


_GATHER_WINDOW = 256


def _sc_gather(table, idx_flat):
    n = idx_flat.shape[1]
    d = table.shape[1]
    mesh = plsc.VectorSubcoreMesh(core_axis_name="c", subcore_axis_name="s")

    @functools.partial(
        pl.kernel,
        out_type=jax.ShapeDtypeStruct((n, d), table.dtype),
        mesh=mesh,
    )
    def gather_kernel(table_hbm, idx_hbm, out_hbm):
        def body(idx_vmem, out_vmem):
            pltpu.sync_copy(table_hbm.at[idx_vmem.at[0]], out_vmem)

        pltpu.emit_pipeline(
            body,
            grid=(n // _GATHER_WINDOW,),
            in_specs=[
                pl.BlockSpec((1, _GATHER_WINDOW), index_map=lambda i: (0, i))
            ],
            out_specs=[
                pl.BlockSpec((_GATHER_WINDOW, d), index_map=lambda i: (i, 0))
            ],
            core_axis_name=("c", "s"),
            dimension_semantics=(pltpu.PARALLEL,),
        )(idx_hbm, out_hbm)

    return gather_kernel(table, idx_flat)


_M_TILE = 1024
_K_TILE = 1280


def _head_kernel(nk, nm, inv_b, tok_ref, pos_ref, w_ref, b_ref, truth_ref,
                 out_ref, loss_ref):
    k = pl.program_id(1)
    emb = (tok_ref[...] + pos_ref[0:1, :]).astype(jnp.bfloat16)
    part = lax.dot_general(
        emb, w_ref[...], (((1,), (1,)), ((), ())),
        preferred_element_type=jnp.float32)

    @pl.when(k == 0)
    def _():
        out_ref[...] = part

    @pl.when(k > 0)
    def _():
        out_ref[...] += part

    @pl.when(k == nk - 1)
    def _():
        logits = out_ref[...] + b_ref[0:1, :]
        out_ref[...] = logits
        mt, v = logits.shape
        mx = jnp.max(logits, axis=1, keepdims=True)
        lse = mx + jnp.log(jnp.sum(jnp.exp(logits - mx), axis=1, keepdims=True))
        lane = lax.broadcasted_iota(jnp.int32, (mt, v), 1)
        tl = jnp.sum(
            jnp.where(lane == truth_ref[...], logits, 0.0),
            axis=1, keepdims=True)
        part_loss = jnp.sum(lse - tl)
        m = pl.program_id(0)

        @pl.when(m == 0)
        def _():
            loss_ref[0, 0] = part_loss

        @pl.when(m > 0)
        def _():
            loss_ref[0, 0] += part_loss

        @pl.when(m == nm - 1)
        def _():
            loss_ref[0, 0] *= inv_b


def _head(tokflat, pos_b, w, b_b, truth2d):
    bsz, kdim = tokflat.shape
    v = w.shape[0]
    nm = bsz // _M_TILE
    nk = kdim // _K_TILE
    out, loss = pl.pallas_call(
        functools.partial(_head_kernel, nk, nm, 1.0 / bsz),
        grid=(nm, nk),
        in_specs=[
            pl.BlockSpec((_M_TILE, _K_TILE), lambda m, k: (m, k)),
            pl.BlockSpec((8, _K_TILE), lambda m, k: (0, k)),
            pl.BlockSpec((v, _K_TILE), lambda m, k: (0, k)),
            pl.BlockSpec((8, v), lambda m, k: (0, 0)),
            pl.BlockSpec((_M_TILE, 1), lambda m, k: (m, 0)),
        ],
        out_specs=[
            pl.BlockSpec((_M_TILE, v), lambda m, k: (m, 0)),
            pl.BlockSpec(
                (1, 1), lambda m, k: (0, 0), memory_space=pltpu.SMEM),
        ],
        out_shape=[
            jax.ShapeDtypeStruct((bsz, v), jnp.float32),
            jax.ShapeDtypeStruct((1, 1), jnp.float32),
        ],
    )(tokflat, pos_b, w, b_b, truth2d)
    return out, loss


def kernel(input_tokens, truth, tok_table, pos_table, W, b):
    bsz, l = input_tokens.shape
    v, d = tok_table.shape
    idx_flat = input_tokens.reshape(1, bsz * l).astype(jnp.int32)
    tokflat = _sc_gather(tok_table, idx_flat).reshape(bsz, l * d)
    pos_b = jnp.broadcast_to(pos_table.reshape(1, l * d), (8, l * d))
    b_b = jnp.broadcast_to(b.reshape(1, v), (8, v))
    truth2d = truth.reshape(bsz, 1).astype(jnp.int32)
    out, loss = _head(tokflat, pos_b, W.astype(jnp.bfloat16), b_b, truth2d)
    return out, loss.reshape(())


# speedup vs baseline: 1.0231x; 1.0231x over previous
import functools

import jax
import jax.numpy as jnp
from jax import lax
from jax.experimental import pallas as pl
from jax.experimental.pallas import tpu as pltpu
from jax.experimental.pallas import tpu_sc as plsc


_GATHER_WINDOW = 256


def _sc_gather(table, idx_flat):
    n = idx_flat.shape[1]
    d = table.shape[1]
    mesh = plsc.VectorSubcoreMesh(core_axis_name="c", subcore_axis_name="s")

    @functools.partial(
        pl.kernel,
        out_type=jax.ShapeDtypeStruct((n, d), table.dtype),
        mesh=mesh,
    )
    def gather_kernel(table_hbm, idx_hbm, out_hbm):
        def body(idx_vmem, out_vmem):
            pltpu.sync_copy(table_hbm.at[idx_vmem.at[0]], out_vmem)

        pltpu.emit_pipeline(
            body,
            grid=(n // _GATHER_WINDOW,),
            in_specs=[
                pl.BlockSpec((1, _GATHER_WINDOW), index_map=lambda i: (0, i))
            ],
            out_specs=[
                pl.BlockSpec((_GATHER_WINDOW, d), index_map=lambda i: (i, 0))
            ],
            core_axis_name=("c", "s"),
            dimension_semantics=(pltpu.PARALLEL,),
        )(idx_hbm, out_hbm)

    return gather_kernel(table, idx_flat)


_M_TILE = 1024
_K_TILE = 2560


def _head_kernel(nk, nm, inv_b, tok_ref, pos_ref, w_ref, b_ref, truth_ref,
                 out_ref, loss_ref):
    k = pl.program_id(1)
    emb = (tok_ref[...] + pos_ref[0:1, :]).astype(jnp.bfloat16)
    part = lax.dot_general(
        emb, w_ref[...], (((1,), (1,)), ((), ())),
        preferred_element_type=jnp.float32)

    @pl.when(k == 0)
    def _():
        out_ref[...] = part

    @pl.when(k > 0)
    def _():
        out_ref[...] += part

    @pl.when(k == nk - 1)
    def _():
        logits = out_ref[...] + b_ref[0:1, :]
        out_ref[...] = logits
        mt, v = logits.shape
        mx = jnp.max(logits, axis=1, keepdims=True)
        lse = mx + jnp.log(jnp.sum(jnp.exp(logits - mx), axis=1, keepdims=True))
        lane = lax.broadcasted_iota(jnp.int32, (mt, v), 1)
        tl = jnp.sum(
            jnp.where(lane == truth_ref[...], logits, 0.0),
            axis=1, keepdims=True)
        part_loss = jnp.sum(lse - tl)
        m = pl.program_id(0)

        @pl.when(m == 0)
        def _():
            loss_ref[0, 0] = part_loss

        @pl.when(m > 0)
        def _():
            loss_ref[0, 0] += part_loss

        @pl.when(m == nm - 1)
        def _():
            loss_ref[0, 0] *= inv_b


def _head(tokflat, pos_b, w, b_b, truth2d):
    bsz, kdim = tokflat.shape
    v = w.shape[0]
    nm = bsz // _M_TILE
    nk = kdim // _K_TILE
    out, loss = pl.pallas_call(
        functools.partial(_head_kernel, nk, nm, 1.0 / bsz),
        grid=(nm, nk),
        in_specs=[
            pl.BlockSpec((_M_TILE, _K_TILE), lambda m, k: (m, k)),
            pl.BlockSpec((8, _K_TILE), lambda m, k: (0, k)),
            pl.BlockSpec((v, _K_TILE), lambda m, k: (0, k)),
            pl.BlockSpec((8, v), lambda m, k: (0, 0)),
            pl.BlockSpec((_M_TILE, 1), lambda m, k: (m, 0)),
        ],
        out_specs=[
            pl.BlockSpec((_M_TILE, v), lambda m, k: (m, 0)),
            pl.BlockSpec(
                (1, 1), lambda m, k: (0, 0), memory_space=pltpu.SMEM),
        ],
        out_shape=[
            jax.ShapeDtypeStruct((bsz, v), jnp.float32),
            jax.ShapeDtypeStruct((1, 1), jnp.float32),
        ],
    )(tokflat, pos_b, w, b_b, truth2d)
    return out, loss


def kernel(input_tokens, truth, tok_table, pos_table, W, b):
    bsz, l = input_tokens.shape
    v, d = tok_table.shape
    idx_flat = input_tokens.reshape(1, bsz * l).astype(jnp.int32)
    tokflat = _sc_gather(tok_table, idx_flat).reshape(bsz, l * d)
    pos_b = jnp.broadcast_to(pos_table.reshape(1, l * d), (8, l * d))
    b_b = jnp.broadcast_to(b.reshape(1, v), (8, v))
    truth2d = truth.reshape(bsz, 1).astype(jnp.int32)
    out, loss = _head(tokflat, pos_b, W.astype(jnp.bfloat16), b_b, truth2d)
    return out, loss.reshape(())


# speedup vs baseline: 1.0498x; 1.0261x over previous
import functools

import jax
import jax.numpy as jnp
from jax import lax
from jax.experimental import pallas as pl
from jax.experimental.pallas import tpu as pltpu
from jax.experimental.pallas import tpu_sc as plsc


_GATHER_WINDOW = 256


def _sc_gather(table, idx_flat):
    n = idx_flat.shape[1]
    d = table.shape[1]
    mesh = plsc.VectorSubcoreMesh(core_axis_name="c", subcore_axis_name="s")

    @functools.partial(
        pl.kernel,
        out_type=jax.ShapeDtypeStruct((n, d), table.dtype),
        mesh=mesh,
    )
    def gather_kernel(table_hbm, idx_hbm, out_hbm):
        def body(idx_vmem, out_vmem):
            pltpu.sync_copy(table_hbm.at[idx_vmem.at[0]], out_vmem)

        pltpu.emit_pipeline(
            body,
            grid=(n // _GATHER_WINDOW,),
            in_specs=[
                pl.BlockSpec((1, _GATHER_WINDOW), index_map=lambda i: (0, i))
            ],
            out_specs=[
                pl.BlockSpec((_GATHER_WINDOW, d), index_map=lambda i: (i, 0))
            ],
            core_axis_name=("c", "s"),
            dimension_semantics=(pltpu.PARALLEL,),
        )(idx_hbm, out_hbm)

    return gather_kernel(table, idx_flat)


_M_TILE = 1024
_K_TILE = 2560


def _head_kernel(nk, nm, tok_ref, pos_ref, w_ref, b_ref, truth_ref,
                 out_ref, loss_ref):
    k = pl.program_id(1)
    emb = (tok_ref[...] + pos_ref[0:1, :]).astype(jnp.bfloat16)
    part = lax.dot_general(
        emb, w_ref[...], (((1,), (1,)), ((), ())),
        preferred_element_type=jnp.float32)

    @pl.when(k == 0)
    def _():
        out_ref[...] = part

    @pl.when(k > 0)
    def _():
        out_ref[...] += part

    @pl.when(k == nk - 1)
    def _():
        logits = out_ref[...] + b_ref[0:1, :]
        out_ref[...] = logits
        mt, v = logits.shape
        mx = jnp.max(logits, axis=1, keepdims=True)
        lse = mx + jnp.log(jnp.sum(jnp.exp(logits - mx), axis=1, keepdims=True))
        lane = lax.broadcasted_iota(jnp.int32, (mt, v), 1)
        tl = jnp.sum(
            jnp.where(lane == truth_ref[...], logits, 0.0),
            axis=1, keepdims=True)
        part_loss = jnp.sum(lse - tl)
        m = pl.program_id(0)

        @pl.when(m == 0)
        def _():
            loss_ref[0, 0] = part_loss

        @pl.when(m > 0)
        def _():
            loss_ref[0, 0] += part_loss


def _head(tokflat, pos_b, w, b_b, truth2d):
    bsz, kdim = tokflat.shape
    v = w.shape[0]
    nm = bsz // _M_TILE
    nk = kdim // _K_TILE
    out, loss = pl.pallas_call(
        functools.partial(_head_kernel, nk, nm),
        grid=(nm, nk),
        in_specs=[
            pl.BlockSpec((_M_TILE, _K_TILE), lambda m, k: (m, k)),
            pl.BlockSpec((8, _K_TILE), lambda m, k: (0, k)),
            pl.BlockSpec((v, _K_TILE), lambda m, k: (0, k)),
            pl.BlockSpec((8, v), lambda m, k: (0, 0)),
            pl.BlockSpec((_M_TILE, 1), lambda m, k: (m, 0)),
        ],
        out_specs=[
            pl.BlockSpec((_M_TILE, v), lambda m, k: (m, 0)),
            pl.BlockSpec(
                (1, 1), lambda m, k: (0, 0), memory_space=pltpu.SMEM),
        ],
        out_shape=[
            jax.ShapeDtypeStruct((bsz, v), jnp.float32),
            jax.ShapeDtypeStruct((1, 1), jnp.float32),
        ],
    )(tokflat, pos_b, w, b_b, truth2d)
    return out, loss


_N_CHUNKS = 2


def kernel(input_tokens, truth, tok_table, pos_table, W, b):
    bsz, l = input_tokens.shape
    v, d = tok_table.shape
    cb = bsz // _N_CHUNKS
    pos_b = jnp.broadcast_to(pos_table.reshape(1, l * d), (8, l * d))
    b_b = jnp.broadcast_to(b.reshape(1, v), (8, v))
    w_bf = W.astype(jnp.bfloat16)
    idx = input_tokens.astype(jnp.int32)
    truth2d = truth.reshape(bsz, 1).astype(jnp.int32)
    outs, loss_sums = [], []
    for c in range(_N_CHUNKS):
        idx_c = idx[c * cb:(c + 1) * cb].reshape(1, cb * l)
        tokflat = _sc_gather(tok_table, idx_c).reshape(cb, l * d)
        out_c, loss_c = _head(
            tokflat, pos_b, w_bf, b_b, truth2d[c * cb:(c + 1) * cb])
        outs.append(out_c)
        loss_sums.append(loss_c[0, 0])
    out = jnp.concatenate(outs, axis=0)
    loss = sum(loss_sums) / bsz
    return out, loss.reshape(())


# speedup vs baseline: 1.0668x; 1.0162x over previous
import functools

import jax
import jax.numpy as jnp
from jax import lax
from jax.experimental import pallas as pl
from jax.experimental.pallas import tpu as pltpu
from jax.experimental.pallas import tpu_sc as plsc


_GATHER_WINDOW = 256


def _sc_gather(table, idx_flat):
    n = idx_flat.shape[1]
    d = table.shape[1]
    mesh = plsc.VectorSubcoreMesh(core_axis_name="c", subcore_axis_name="s")

    @functools.partial(
        pl.kernel,
        out_type=jax.ShapeDtypeStruct((n, d), table.dtype),
        mesh=mesh,
    )
    def gather_kernel(table_hbm, idx_hbm, out_hbm):
        def body(idx_vmem, out_vmem):
            pltpu.sync_copy(table_hbm.at[idx_vmem.at[0]], out_vmem)

        pltpu.emit_pipeline(
            body,
            grid=(n // _GATHER_WINDOW,),
            in_specs=[
                pl.BlockSpec((1, _GATHER_WINDOW), index_map=lambda i: (0, i))
            ],
            out_specs=[
                pl.BlockSpec((_GATHER_WINDOW, d), index_map=lambda i: (i, 0))
            ],
            core_axis_name=("c", "s"),
            dimension_semantics=(pltpu.PARALLEL,),
        )(idx_hbm, out_hbm)

    return gather_kernel(table, idx_flat)


_M_TILE = 1024
_K_TILE = 2560


def _head_kernel(nk, nm, tok_ref, pos_ref, w_ref, b_ref, truth_ref,
                 out_ref, loss_ref):
    k = pl.program_id(1)
    emb = (tok_ref[...] + pos_ref[0:1, :]).astype(jnp.bfloat16)
    part = lax.dot_general(
        emb, w_ref[...], (((1,), (1,)), ((), ())),
        preferred_element_type=jnp.float32)

    @pl.when(k == 0)
    def _():
        out_ref[...] = part

    @pl.when(k > 0)
    def _():
        out_ref[...] += part

    @pl.when(k == nk - 1)
    def _():
        logits = out_ref[...] + b_ref[0:1, :]
        out_ref[...] = logits
        mt, v = logits.shape
        mx = jnp.max(logits, axis=1, keepdims=True)
        lse = mx + jnp.log(jnp.sum(jnp.exp(logits - mx), axis=1, keepdims=True))
        lane = lax.broadcasted_iota(jnp.int32, (mt, v), 1)
        tl = jnp.sum(
            jnp.where(lane == truth_ref[...], logits, 0.0),
            axis=1, keepdims=True)
        part_loss = jnp.sum(lse - tl)
        m = pl.program_id(0)

        @pl.when(m == 0)
        def _():
            loss_ref[0, 0] = part_loss

        @pl.when(m > 0)
        def _():
            loss_ref[0, 0] += part_loss


def _head(tokflat, pos_b, w, b_b, truth2d):
    bsz, kdim = tokflat.shape
    v = w.shape[0]
    nm = bsz // _M_TILE
    nk = kdim // _K_TILE
    out, loss = pl.pallas_call(
        functools.partial(_head_kernel, nk, nm),
        grid=(nm, nk),
        in_specs=[
            pl.BlockSpec((_M_TILE, _K_TILE), lambda m, k: (m, k)),
            pl.BlockSpec((8, _K_TILE), lambda m, k: (0, k)),
            pl.BlockSpec((v, _K_TILE), lambda m, k: (0, k)),
            pl.BlockSpec((8, v), lambda m, k: (0, 0)),
            pl.BlockSpec((_M_TILE, 1), lambda m, k: (m, 0)),
        ],
        out_specs=[
            pl.BlockSpec((_M_TILE, v), lambda m, k: (m, 0)),
            pl.BlockSpec(
                (1, 1), lambda m, k: (0, 0), memory_space=pltpu.SMEM),
        ],
        out_shape=[
            jax.ShapeDtypeStruct((bsz, v), jnp.float32),
            jax.ShapeDtypeStruct((1, 1), jnp.float32),
        ],
    )(tokflat, pos_b, w, b_b, truth2d)
    return out, loss


_N_CHUNKS = 4


def kernel(input_tokens, truth, tok_table, pos_table, W, b):
    bsz, l = input_tokens.shape
    v, d = tok_table.shape
    cb = bsz // _N_CHUNKS
    pos_b = jnp.broadcast_to(pos_table.reshape(1, l * d), (8, l * d))
    b_b = jnp.broadcast_to(b.reshape(1, v), (8, v))
    w_bf = W.astype(jnp.bfloat16)
    idx = input_tokens.astype(jnp.int32)
    truth2d = truth.reshape(bsz, 1).astype(jnp.int32)
    tokflats = [
        _sc_gather(
            tok_table,
            idx[c * cb:(c + 1) * cb].reshape(1, cb * l)).reshape(cb, l * d)
        for c in range(_N_CHUNKS)
    ]
    outs, loss_sums = [], []
    for c in range(_N_CHUNKS):
        out_c, loss_c = _head(
            tokflats[c], pos_b, w_bf, b_b, truth2d[c * cb:(c + 1) * cb])
        outs.append(out_c)
        loss_sums.append(loss_c[0, 0])
    out = jnp.concatenate(outs, axis=0)
    loss = sum(loss_sums) / bsz
    return out, loss.reshape(())


# speedup vs baseline: 1.4479x; 1.3572x over previous
import functools

import jax
import jax.numpy as jnp
from jax import lax
from jax.experimental import pallas as pl
from jax.experimental.pallas import tpu as pltpu
from jax.experimental.pallas import tpu_sc as plsc


_GATHER_WINDOW = 256


def _sc_gather(table, idx_flat):
    n = idx_flat.shape[1]
    d = table.shape[1]
    mesh = plsc.VectorSubcoreMesh(core_axis_name="c", subcore_axis_name="s")

    @functools.partial(
        pl.kernel,
        out_type=jax.ShapeDtypeStruct((n, d), table.dtype),
        mesh=mesh,
    )
    def gather_kernel(table_hbm, idx_hbm, out_hbm):
        def body(idx_vmem, out_vmem):
            pltpu.sync_copy(table_hbm.at[idx_vmem.at[0]], out_vmem)

        pltpu.emit_pipeline(
            body,
            grid=(n // _GATHER_WINDOW,),
            in_specs=[
                pl.BlockSpec((1, _GATHER_WINDOW), index_map=lambda i: (0, i))
            ],
            out_specs=[
                pl.BlockSpec((_GATHER_WINDOW, d), index_map=lambda i: (i, 0))
            ],
            core_axis_name=("c", "s"),
            dimension_semantics=(pltpu.PARALLEL,),
        )(idx_hbm, out_hbm)

    return gather_kernel(table, idx_flat)


_M_TILE = 1024
_P_CHUNK = 8


def _head_kernel(nk, nm, tok_ref, pos_ref, w_ref, b_ref, truth_ref,
                 out_ref, loss_ref):
    k = pl.program_id(1)
    pc = tok_ref.shape[0]
    emb = jnp.concatenate(
        [(tok_ref[j] + pos_ref[j]).astype(jnp.bfloat16) for j in range(pc)],
        axis=1)
    wmat = w_ref[...].reshape(pc * w_ref.shape[1], w_ref.shape[2])
    part = lax.dot_general(
        emb, wmat, (((1,), (0,)), ((), ())),
        preferred_element_type=jnp.float32)

    @pl.when(k == 0)
    def _():
        out_ref[...] = part

    @pl.when(k > 0)
    def _():
        out_ref[...] += part

    @pl.when(k == nk - 1)
    def _():
        logits = out_ref[...] + b_ref[0:1, :]
        out_ref[...] = logits
        mt, v = logits.shape
        mx = jnp.max(logits, axis=1, keepdims=True)
        lse = mx + jnp.log(jnp.sum(jnp.exp(logits - mx), axis=1, keepdims=True))
        lane = lax.broadcasted_iota(jnp.int32, (mt, v), 1)
        tl = jnp.sum(
            jnp.where(lane == truth_ref[...], logits, 0.0),
            axis=1, keepdims=True)
        part_loss = jnp.sum(lse - tl)
        m = pl.program_id(0)

        @pl.when(m == 0)
        def _():
            loss_ref[0, 0] = part_loss

        @pl.when(m > 0)
        def _():
            loss_ref[0, 0] += part_loss


def _head(tok3, pos, w3, b_b, truth2d):
    l, cb, d = tok3.shape
    v = w3.shape[2]
    nm = cb // _M_TILE
    nk = l // _P_CHUNK
    out, loss = pl.pallas_call(
        functools.partial(_head_kernel, nk, nm),
        grid=(nm, nk),
        in_specs=[
            pl.BlockSpec((_P_CHUNK, _M_TILE, d), lambda m, k: (k, m, 0)),
            pl.BlockSpec((_P_CHUNK, d), lambda m, k: (k, 0)),
            pl.BlockSpec((_P_CHUNK, d, v), lambda m, k: (k, 0, 0)),
            pl.BlockSpec((8, v), lambda m, k: (0, 0)),
            pl.BlockSpec((_M_TILE, 1), lambda m, k: (m, 0)),
        ],
        out_specs=[
            pl.BlockSpec((_M_TILE, v), lambda m, k: (m, 0)),
            pl.BlockSpec(
                (1, 1), lambda m, k: (0, 0), memory_space=pltpu.SMEM),
        ],
        out_shape=[
            jax.ShapeDtypeStruct((cb, v), jnp.float32),
            jax.ShapeDtypeStruct((1, 1), jnp.float32),
        ],
    )(tok3, pos, w3, b_b, truth2d)
    return out, loss


_N_CHUNKS = 4


def kernel(input_tokens, truth, tok_table, pos_table, W, b):
    bsz, l = input_tokens.shape
    v, d = tok_table.shape
    cb = bsz // _N_CHUNKS
    w3 = W.astype(jnp.bfloat16).reshape(v, l, d).transpose(1, 2, 0)
    b_b = jnp.broadcast_to(b.reshape(1, v), (8, v))
    idx_t = input_tokens.astype(jnp.int32).T
    truth2d = truth.reshape(bsz, 1).astype(jnp.int32)
    tok3s = [
        _sc_gather(
            tok_table,
            idx_t[:, c * cb:(c + 1) * cb].reshape(1, l * cb)).reshape(l, cb, d)
        for c in range(_N_CHUNKS)
    ]
    outs, loss_sums = [], []
    for c in range(_N_CHUNKS):
        out_c, loss_c = _head(
            tok3s[c], pos_table, w3, b_b, truth2d[c * cb:(c + 1) * cb])
        outs.append(out_c)
        loss_sums.append(loss_c[0, 0])
    out = jnp.concatenate(outs, axis=0)
    loss = sum(loss_sums) / bsz
    return out, loss.reshape(())
